# Initial kernel scaffold; baseline (speedup 1.0000x reference)
import jax
import jax.numpy as jnp
from jax.experimental import pallas as pl


def kernel(x, edge_index, W_pre, b_pre, W1, b1, W2, b2, W3, b3, W_head, b_head):
    raise NotImplementedError("write your pallas kernel here")


# speedup vs baseline: 6.5087x; 6.5087x over previous
import functools

import jax
import jax.numpy as jnp
from jax import lax
from jax.experimental import pallas as pl
from jax.experimental.pallas import tpu as pltpu
from jax.experimental.pallas import tpu_sc as plsc

NCORE = 2
NSUB = 16
CHUNK = 128
BR = 1000


def _cdiv(a, b):
    return (a + b - 1) // b


def _splits(n):
    rpw0 = (n // NSUB) // 8 * 8
    rem = n - NSUB * rpw0
    assert rem % 8 == 0
    return rpw0, rem


@functools.lru_cache(maxsize=None)
def _make_agg(n, nch):
    mesh = plsc.VectorSubcoreMesh(
        core_axis_name="c", subcore_axis_name="s",
        num_cores=NCORE, num_subcores=NSUB)
    rpw0, rem = _splits(n)
    ZB = CHUNK
    zrows = ((n + 16 + NSUB - 1) // NSUB + 7) // 8 * 8
    npad = NSUB * zrows
    assert npad >= n + 16

    @functools.partial(
        pl.kernel,
        out_type=jax.ShapeDtypeStruct((NCORE * n, 128), jnp.float32),
        mesh=mesh,
        scratch_types=[
            pltpu.VMEM((nch, CHUNK), jnp.int32),
            pltpu.VMEM((nch, CHUNK), jnp.int32),
            pltpu.VMEM((CHUNK, 128), jnp.float32),
            pltpu.VMEM_SHARED((npad, 128), jnp.float32),
            pltpu.SemaphoreType.DMA,
        ],
    )
    def agg_kernel(g_hbm, src_hbm, dst_hbm, zeros_hbm, out_hbm,
                   sidx, didx, rows, acc, sem):
        c = lax.axis_index("c")
        s = lax.axis_index("s")
        pltpu.sync_copy(zeros_hbm, rows)
        for off in range(0, zrows, ZB):
            sz = min(ZB, zrows - off)
            pltpu.sync_copy(rows.at[pl.ds(0, sz)],
                            acc.at[pl.ds(s * zrows + off, sz)])
        pltpu.sync_copy(src_hbm.at[c, s], sidx)
        pltpu.sync_copy(dst_hbm.at[s], didx)
        plsc.subcore_barrier()

        def body(j, carry):
            pltpu.async_copy(g_hbm.at[sidx.at[j]], rows, sem).wait()
            pltpu.sync_copy(rows, acc.at[didx.at[j]], add=True)
            return carry

        lax.fori_loop(0, nch, body, 0)
        plsc.subcore_barrier()
        base = c * n + s * rpw0
        for off in range(0, rpw0, ZB):
            sz = min(ZB, rpw0 - off)
            pltpu.sync_copy(acc.at[pl.ds(s * rpw0 + off, sz)],
                            rows.at[pl.ds(0, sz)])
            pltpu.sync_copy(rows.at[pl.ds(0, sz)],
                            out_hbm.at[pl.ds(base + off, sz)])
        if rem:
            @pl.when(s == NSUB - 1)
            def _():
                pltpu.sync_copy(acc.at[pl.ds(NSUB * rpw0, rem)],
                                rows.at[pl.ds(0, rem)])
                pltpu.sync_copy(rows.at[pl.ds(0, rem)],
                                out_hbm.at[pl.ds(c * n + NSUB * rpw0, rem)])

    return agg_kernel


def _dinv_of(deg_ref):
    return lax.rsqrt(deg_ref[0, :, 0:1])


def _k01_body(x_ref, wpre_ref, bpre_ref, deg_ref, w1_ref, h_ref, g_ref):
    h = jnp.maximum(
        jnp.dot(x_ref[...], wpre_ref[...], preferred_element_type=jnp.float32)
        + bpre_ref[...], 0.0)
    h_ref[...] = h
    dinv = _dinv_of(deg_ref)
    g = jnp.dot(h, w1_ref[...], preferred_element_type=jnp.float32) * dinv
    g_ref[0] = g[:, :128]
    g_ref[1] = g[:, 128:]


def _kmid_body(h_ref, acc_ref, deg_ref, b_ref, w_ref, hn_ref, g_ref):
    dinv = _dinv_of(deg_ref)
    agg = jnp.concatenate([acc_ref[0], acc_ref[1]], axis=1)
    hn = h_ref[...] + jnp.maximum(agg * dinv + b_ref[...], 0.0)
    hn_ref[...] = hn
    g = jnp.dot(hn, w_ref[...], preferred_element_type=jnp.float32) * dinv
    g_ref[0] = g[:, :128]
    g_ref[1] = g[:, 128:]


def _kfin_body(h_ref, acc_ref, deg_ref, b_ref, wh_ref, bh_ref, y_ref):
    dinv = _dinv_of(deg_ref)
    agg = jnp.concatenate([acc_ref[0], acc_ref[1]], axis=1)
    hn = h_ref[...] + jnp.maximum(agg * dinv + b_ref[...], 0.0)
    y_ref[...] = (
        jnp.dot(hn, wh_ref[...], preferred_element_type=jnp.float32)
        + bh_ref[...])


def _row_spec(d):
    return pl.BlockSpec((BR, d), lambda i: (i, 0))


def _full_spec(shape):
    nd = len(shape)
    return pl.BlockSpec(shape, lambda i, _n=nd: (0,) * _n)


def _split_spec():
    return pl.BlockSpec((2, BR, 128), lambda i: (0, i, 0))


def _deg_spec():
    return pl.BlockSpec((2, BR, 128), lambda i: (0, i, 0))


def kernel(x, edge_index, W_pre, b_pre, W1, b1, W2, b2, W3, b3, W_head, b_head):
    n, d = x.shape
    e = edge_index.shape[1]
    dout = W_head.shape[1]
    grid = (n // BR,)

    src = edge_index[0]
    dst = edge_index[1]

    loop = jnp.arange(n, dtype=src.dtype)
    srcl = jnp.concatenate([src, loop])
    dstl = jnp.concatenate([dst, loop])
    e2 = e + n
    nch = _cdiv(e2, NSUB * CHUNK)
    cap = NSUB * CHUNK * nch
    dsta = jnp.pad(dstl, (0, cap - e2), constant_values=n)
    dsta = dsta.reshape(NSUB, nch, CHUNK)
    srcp = jnp.pad(srcl, (0, cap - e2)).reshape(NSUB, nch, CHUNK)
    srca = jnp.stack([srcp, srcp + n], axis=0)

    zeros128 = jnp.zeros((CHUNK, 128), jnp.float32)

    agg = _make_agg(n, nch)

    ones2n = jnp.ones((NCORE * n, 128), jnp.float32)
    deg = agg(ones2n, srca, dsta, zeros128).reshape(NCORE, n, 128)

    b_pre2 = b_pre.reshape(1, d)
    b1_2 = b1.reshape(1, d)
    b2_2 = b2.reshape(1, d)
    b3_2 = b3.reshape(1, d)
    bh_2 = b_head.reshape(1, dout)

    k01 = pl.pallas_call(
        _k01_body,
        grid=grid,
        in_specs=[_row_spec(d), _full_spec((d, d)), _full_spec((1, d)),
                  _deg_spec(), _full_spec((d, d))],
        out_specs=[_row_spec(d), _split_spec()],
        out_shape=[jax.ShapeDtypeStruct((n, d), jnp.float32),
                   jax.ShapeDtypeStruct((2, n, 128), jnp.float32)],
    )
    kmid = pl.pallas_call(
        _kmid_body,
        grid=grid,
        in_specs=[_row_spec(d), _split_spec(), _deg_spec(),
                  _full_spec((1, d)), _full_spec((d, d))],
        out_specs=[_row_spec(d), _split_spec()],
        out_shape=[jax.ShapeDtypeStruct((n, d), jnp.float32),
                   jax.ShapeDtypeStruct((2, n, 128), jnp.float32)],
    )
    kfin = pl.pallas_call(
        _kfin_body,
        grid=grid,
        in_specs=[_row_spec(d), _split_spec(), _deg_spec(),
                  _full_spec((1, d)), _full_spec((d, dout)),
                  _full_spec((1, dout))],
        out_specs=_row_spec(dout),
        out_shape=jax.ShapeDtypeStruct((n, dout), jnp.float32),
    )

    h0, g1 = k01(x, W_pre, b_pre2, deg, W1)
    acc1 = agg(g1.reshape(2 * n, 128), srca, dsta, zeros128).reshape(2, n, 128)
    h1, g2 = kmid(h0, acc1, deg, b1_2, W2)
    acc2 = agg(g2.reshape(2 * n, 128), srca, dsta, zeros128).reshape(2, n, 128)
    h2, g3 = kmid(h1, acc2, deg, b2_2, W3)
    acc3 = agg(g3.reshape(2 * n, 128), srca, dsta, zeros128).reshape(2, n, 128)
    return kfin(h2, acc3, deg, b3_2, W_head, bh_2)


# speedup vs baseline: 7.3563x; 1.1302x over previous
import functools

import jax
import jax.numpy as jnp
from jax import lax
from jax.experimental import pallas as pl
from jax.experimental.pallas import tpu as pltpu
from jax.experimental.pallas import tpu_sc as plsc

NCORE = 2
NSUB = 16
CHUNK = 112
BLK = 8
BR = 1000


def _cdiv(a, b):
    return (a + b - 1) // b


def _splits(n):
    rpw0 = (n // NSUB) // 8 * 8
    rem = n - NSUB * rpw0
    assert rem % 8 == 0
    return rpw0, rem


@functools.lru_cache(maxsize=None)
def _make_agg(n, nch):
    mesh = plsc.VectorSubcoreMesh(
        core_axis_name="c", subcore_axis_name="s",
        num_cores=NCORE, num_subcores=NSUB)
    rpw0, rem = _splits(n)
    assert nch % (2 * BLK) == 0
    nbp = nch // (2 * BLK)
    npad = (n + 16 + 7) // 8 * 8
    zrem = npad - NSUB * rpw0

    @functools.partial(
        pl.kernel,
        out_type=jax.ShapeDtypeStruct((NCORE * n, 128), jnp.float32),
        mesh=mesh,
        scratch_types=[
            pltpu.VMEM((nch, CHUNK), jnp.int32),
            pltpu.VMEM((BLK, CHUNK), jnp.int32),
            pltpu.VMEM((BLK, CHUNK), jnp.int32),
            pltpu.VMEM((CHUNK, 128), jnp.float32),
            pltpu.VMEM((CHUNK, 128), jnp.float32),
            pltpu.VMEM_SHARED((npad, 128), jnp.float32),
            pltpu.SemaphoreType.DMA,
            pltpu.SemaphoreType.DMA,
            pltpu.SemaphoreType.DMA,
            pltpu.SemaphoreType.DMA,
        ],
    )
    def agg_kernel(g_hbm, src_hbm, dst_hbm, zeros_hbm, out_hbm,
                   sidx, blk_a, blk_b, rows_a, rows_b, acc,
                   sem_a, sem_b, sem_da, sem_db):
        c = lax.axis_index("c")
        s = lax.axis_index("s")
        pltpu.sync_copy(zeros_hbm, rows_a)
        for off in range(0, rpw0, CHUNK):
            sz = min(CHUNK, rpw0 - off)
            pltpu.sync_copy(rows_a.at[pl.ds(0, sz)],
                            acc.at[pl.ds(s * rpw0 + off, sz)])
        if zrem:
            @pl.when(s == NSUB - 1)
            def _():
                for off in range(0, zrem, CHUNK):
                    sz = min(CHUNK, zrem - off)
                    pltpu.sync_copy(rows_a.at[pl.ds(0, sz)],
                                    acc.at[pl.ds(NSUB * rpw0 + off, sz)])
        pltpu.sync_copy(src_hbm.at[c, s], sidx)
        plsc.subcore_barrier()

        rows = (rows_a, rows_b)
        sems = (sem_a, sem_b)

        def gather(j, parity):
            pltpu.async_copy(g_hbm.at[sidx.at[j]], rows[parity], sems[parity])

        def wait_rows(parity):
            pltpu.make_async_copy(
                g_hbm.at[pl.ds(0, CHUNK)], rows[parity], sems[parity]).wait()

        def fetch_blk(bi, blk, sem):
            off = pl.multiple_of(bi * BLK, BLK)
            pltpu.async_copy(dst_hbm.at[s].at[pl.ds(off, BLK)], blk, sem)

        def wait_blk(blk, sem):
            pltpu.make_async_copy(
                dst_hbm.at[0].at[pl.ds(0, BLK)], blk, sem).wait()

        nb = nch // BLK
        fetch_blk(0, blk_a, sem_da)
        fetch_blk(1, blk_b, sem_db)
        gather(0, 0)

        def body(p, carry):
            j0 = 2 * BLK * p
            wait_blk(blk_a, sem_da)
            for k in range(BLK):
                j = j0 + k
                gather(jnp.minimum(j + 1, nch - 1), (k + 1) % 2)
                wait_rows(k % 2)
                pltpu.sync_copy(rows[k % 2], acc.at[blk_a.at[k]], add=True)
            fetch_blk(jnp.minimum(2 * p + 2, nb - 1), blk_a, sem_da)
            wait_blk(blk_b, sem_db)
            for k in range(BLK):
                j = j0 + BLK + k
                gather(jnp.minimum(j + 1, nch - 1), (k + 1) % 2)
                wait_rows(k % 2)
                pltpu.sync_copy(rows[k % 2], acc.at[blk_b.at[k]], add=True)
            fetch_blk(jnp.minimum(2 * p + 3, nb - 1), blk_b, sem_db)
            return carry

        lax.fori_loop(0, nbp, body, 0)
        wait_rows(0)
        wait_blk(blk_a, sem_da)
        wait_blk(blk_b, sem_db)
        plsc.subcore_barrier()
        base = c * n + s * rpw0
        for off in range(0, rpw0, CHUNK):
            sz = min(CHUNK, rpw0 - off)
            pltpu.sync_copy(acc.at[pl.ds(s * rpw0 + off, sz)],
                            rows_a.at[pl.ds(0, sz)])
            pltpu.sync_copy(rows_a.at[pl.ds(0, sz)],
                            out_hbm.at[pl.ds(base + off, sz)])
        if rem:
            @pl.when(s == NSUB - 1)
            def _():
                pltpu.sync_copy(acc.at[pl.ds(NSUB * rpw0, rem)],
                                rows_a.at[pl.ds(0, rem)])
                pltpu.sync_copy(rows_a.at[pl.ds(0, rem)],
                                out_hbm.at[pl.ds(c * n + NSUB * rpw0, rem)])

    return agg_kernel


def _dinv_of(deg_ref):
    return lax.rsqrt(deg_ref[0, :, 0:1])


def _k01_body(x_ref, wpre_ref, bpre_ref, deg_ref, w1_ref, h_ref, g_ref):
    h = jnp.maximum(
        jnp.dot(x_ref[...], wpre_ref[...], preferred_element_type=jnp.float32)
        + bpre_ref[...], 0.0)
    h_ref[...] = h
    dinv = _dinv_of(deg_ref)
    g = jnp.dot(h, w1_ref[...], preferred_element_type=jnp.float32) * dinv
    g_ref[0] = g[:, :128]
    g_ref[1] = g[:, 128:]


def _kmid_body(h_ref, acc_ref, deg_ref, b_ref, w_ref, hn_ref, g_ref):
    dinv = _dinv_of(deg_ref)
    agg = jnp.concatenate([acc_ref[0], acc_ref[1]], axis=1)
    hn = h_ref[...] + jnp.maximum(agg * dinv + b_ref[...], 0.0)
    hn_ref[...] = hn
    g = jnp.dot(hn, w_ref[...], preferred_element_type=jnp.float32) * dinv
    g_ref[0] = g[:, :128]
    g_ref[1] = g[:, 128:]


def _kfin_body(h_ref, acc_ref, deg_ref, b_ref, wh_ref, bh_ref, y_ref):
    dinv = _dinv_of(deg_ref)
    agg = jnp.concatenate([acc_ref[0], acc_ref[1]], axis=1)
    hn = h_ref[...] + jnp.maximum(agg * dinv + b_ref[...], 0.0)
    y_ref[...] = (
        jnp.dot(hn, wh_ref[...], preferred_element_type=jnp.float32)
        + bh_ref[...])


def _row_spec(d):
    return pl.BlockSpec((BR, d), lambda i: (i, 0))


def _full_spec(shape):
    nd = len(shape)
    return pl.BlockSpec(shape, lambda i, _n=nd: (0,) * _n)


def _split_spec():
    return pl.BlockSpec((2, BR, 128), lambda i: (0, i, 0))


def _deg_spec():
    return pl.BlockSpec((2, BR, 128), lambda i: (0, i, 0))


def kernel(x, edge_index, W_pre, b_pre, W1, b1, W2, b2, W3, b3, W_head, b_head):
    n, d = x.shape
    e = edge_index.shape[1]
    dout = W_head.shape[1]
    grid = (n // BR,)

    src = edge_index[0]
    dst = edge_index[1]

    loop = jnp.arange(n, dtype=src.dtype)
    srcl = jnp.concatenate([src, loop])
    dstl = jnp.concatenate([dst, loop])
    e2 = e + n
    nch = _cdiv(_cdiv(e2, NSUB * CHUNK), 2 * BLK) * 2 * BLK
    cap = NSUB * CHUNK * nch
    dsta = jnp.pad(dstl, (0, cap - e2), constant_values=n)
    dsta = dsta.reshape(NSUB, nch, CHUNK)
    srcp = jnp.pad(srcl, (0, cap - e2)).reshape(NSUB, nch, CHUNK)
    srca = jnp.stack([srcp, srcp + n], axis=0)

    zeros128 = jnp.zeros((CHUNK, 128), jnp.float32)

    agg = _make_agg(n, nch)

    ones2n = jnp.ones((NCORE * n, 128), jnp.float32)
    deg = agg(ones2n, srca, dsta, zeros128).reshape(NCORE, n, 128)

    b_pre2 = b_pre.reshape(1, d)
    b1_2 = b1.reshape(1, d)
    b2_2 = b2.reshape(1, d)
    b3_2 = b3.reshape(1, d)
    bh_2 = b_head.reshape(1, dout)

    k01 = pl.pallas_call(
        _k01_body,
        grid=grid,
        in_specs=[_row_spec(d), _full_spec((d, d)), _full_spec((1, d)),
                  _deg_spec(), _full_spec((d, d))],
        out_specs=[_row_spec(d), _split_spec()],
        out_shape=[jax.ShapeDtypeStruct((n, d), jnp.float32),
                   jax.ShapeDtypeStruct((2, n, 128), jnp.float32)],
    )
    kmid = pl.pallas_call(
        _kmid_body,
        grid=grid,
        in_specs=[_row_spec(d), _split_spec(), _deg_spec(),
                  _full_spec((1, d)), _full_spec((d, d))],
        out_specs=[_row_spec(d), _split_spec()],
        out_shape=[jax.ShapeDtypeStruct((n, d), jnp.float32),
                   jax.ShapeDtypeStruct((2, n, 128), jnp.float32)],
    )
    kfin = pl.pallas_call(
        _kfin_body,
        grid=grid,
        in_specs=[_row_spec(d), _split_spec(), _deg_spec(),
                  _full_spec((1, d)), _full_spec((d, dout)),
                  _full_spec((1, dout))],
        out_specs=_row_spec(dout),
        out_shape=jax.ShapeDtypeStruct((n, dout), jnp.float32),
    )

    h0, g1 = k01(x, W_pre, b_pre2, deg, W1)
    acc1 = agg(g1.reshape(2 * n, 128), srca, dsta, zeros128).reshape(2, n, 128)
    h1, g2 = kmid(h0, acc1, deg, b1_2, W2)
    acc2 = agg(g2.reshape(2 * n, 128), srca, dsta, zeros128).reshape(2, n, 128)
    h2, g3 = kmid(h1, acc2, deg, b2_2, W3)
    acc3 = agg(g3.reshape(2 * n, 128), srca, dsta, zeros128).reshape(2, n, 128)
    return kfin(h2, acc3, deg, b3_2, W_head, bh_2)


# speedup vs baseline: 9.1890x; 1.2491x over previous
import functools

import jax
import jax.numpy as jnp
from jax import lax
from jax.experimental import pallas as pl
from jax.experimental.pallas import tpu as pltpu
from jax.experimental.pallas import tpu_sc as plsc

NCORE = 2
NSUB = 16
CHUNK = 112
BLK = 8
BR = 1000


def _cdiv(a, b):
    return (a + b - 1) // b


def _splits(n):
    rpw0 = (n // NSUB) // 8 * 8
    rem = n - NSUB * rpw0
    assert rem % 8 == 0
    return rpw0, rem


@functools.lru_cache(maxsize=None)
def _make_deg(n, nch):
    mesh = plsc.VectorSubcoreMesh(
        core_axis_name="c", subcore_axis_name="s",
        num_cores=NCORE, num_subcores=NSUB)
    rpw0, rem = _splits(n)
    assert nch % (2 * BLK) == 0
    nbp = nch // (2 * BLK)
    npad = (n + 16 + 7) // 8 * 8
    zrem = npad - NSUB * rpw0

    @functools.partial(
        pl.kernel,
        out_type=jax.ShapeDtypeStruct((NCORE * n, 128), jnp.float32),
        mesh=mesh,
        scratch_types=[
            pltpu.VMEM((BLK, CHUNK), jnp.int32),
            pltpu.VMEM((BLK, CHUNK), jnp.int32),
            pltpu.VMEM((CHUNK, 128), jnp.float32),
            pltpu.VMEM_SHARED((npad, 128), jnp.float32),
            pltpu.SemaphoreType.DMA,
            pltpu.SemaphoreType.DMA,
        ],
    )
    def deg_kernel(dst_hbm, ones_hbm, zeros_hbm, out_hbm,
                   blk_a, blk_b, ones_v, acc, sem_da, sem_db):
        c = lax.axis_index("c")
        s = lax.axis_index("s")
        pltpu.sync_copy(zeros_hbm, ones_v)
        for off in range(0, rpw0, CHUNK):
            sz = min(CHUNK, rpw0 - off)
            pltpu.sync_copy(ones_v.at[pl.ds(0, sz)],
                            acc.at[pl.ds(s * rpw0 + off, sz)])
        if zrem:
            @pl.when(s == NSUB - 1)
            def _():
                for off in range(0, zrem, CHUNK):
                    sz = min(CHUNK, zrem - off)
                    pltpu.sync_copy(ones_v.at[pl.ds(0, sz)],
                                    acc.at[pl.ds(NSUB * rpw0 + off, sz)])
        pltpu.sync_copy(ones_hbm, ones_v)
        plsc.subcore_barrier()

        def fetch_blk(bi, blk, sem):
            off = pl.multiple_of(bi * BLK, BLK)
            pltpu.async_copy(dst_hbm.at[c, s].at[pl.ds(off, BLK)], blk, sem)

        def wait_blk(blk, sem):
            pltpu.make_async_copy(
                dst_hbm.at[0, 0].at[pl.ds(0, BLK)], blk, sem).wait()

        nb = nch // BLK
        fetch_blk(0, blk_a, sem_da)
        fetch_blk(1, blk_b, sem_db)

        def body(p, carry):
            wait_blk(blk_a, sem_da)
            for k in range(BLK):
                pltpu.sync_copy(ones_v, acc.at[blk_a.at[k]], add=True)
            fetch_blk(jnp.minimum(2 * p + 2, nb - 1), blk_a, sem_da)
            wait_blk(blk_b, sem_db)
            for k in range(BLK):
                pltpu.sync_copy(ones_v, acc.at[blk_b.at[k]], add=True)
            fetch_blk(jnp.minimum(2 * p + 3, nb - 1), blk_b, sem_db)
            return carry

        lax.fori_loop(0, nbp, body, 0)
        wait_blk(blk_a, sem_da)
        wait_blk(blk_b, sem_db)
        plsc.subcore_barrier()
        base = c * n + s * rpw0
        for off in range(0, rpw0, CHUNK):
            sz = min(CHUNK, rpw0 - off)
            pltpu.sync_copy(acc.at[pl.ds(s * rpw0 + off, sz)],
                            ones_v.at[pl.ds(0, sz)])
            pltpu.sync_copy(ones_v.at[pl.ds(0, sz)],
                            out_hbm.at[pl.ds(base + off, sz)])
        if rem:
            @pl.when(s == NSUB - 1)
            def _():
                pltpu.sync_copy(acc.at[pl.ds(NSUB * rpw0, rem)],
                                ones_v.at[pl.ds(0, rem)])
                pltpu.sync_copy(ones_v.at[pl.ds(0, rem)],
                                out_hbm.at[pl.ds(c * n + NSUB * rpw0, rem)])

    return deg_kernel


@functools.lru_cache(maxsize=None)
def _make_agg(n, nch):
    mesh = plsc.VectorSubcoreMesh(
        core_axis_name="c", subcore_axis_name="s",
        num_cores=NCORE, num_subcores=NSUB)
    rpw0, rem = _splits(n)
    assert nch % (2 * BLK) == 0
    nbp = nch // (2 * BLK)
    npad = (n + 16 + 7) // 8 * 8
    zrem = npad - NSUB * rpw0

    @functools.partial(
        pl.kernel,
        out_type=jax.ShapeDtypeStruct((NCORE * n, 128), jnp.float32),
        mesh=mesh,
        scratch_types=[
            pltpu.VMEM((nch, CHUNK), jnp.int32),
            pltpu.VMEM((BLK, CHUNK), jnp.int32),
            pltpu.VMEM((BLK, CHUNK), jnp.int32),
            pltpu.VMEM((CHUNK, 128), jnp.float32),
            pltpu.VMEM((CHUNK, 128), jnp.float32),
            pltpu.VMEM_SHARED((npad, 128), jnp.float32),
            pltpu.SemaphoreType.DMA,
            pltpu.SemaphoreType.DMA,
            pltpu.SemaphoreType.DMA,
            pltpu.SemaphoreType.DMA,
        ],
    )
    def agg_kernel(g_hbm, src_hbm, dst_hbm, zeros_hbm, out_hbm,
                   sidx, blk_a, blk_b, rows_a, rows_b, acc,
                   sem_a, sem_b, sem_da, sem_db):
        c = lax.axis_index("c")
        s = lax.axis_index("s")
        pltpu.sync_copy(zeros_hbm, rows_a)
        for off in range(0, rpw0, CHUNK):
            sz = min(CHUNK, rpw0 - off)
            pltpu.sync_copy(rows_a.at[pl.ds(0, sz)],
                            acc.at[pl.ds(s * rpw0 + off, sz)])
        if zrem:
            @pl.when(s == NSUB - 1)
            def _():
                for off in range(0, zrem, CHUNK):
                    sz = min(CHUNK, zrem - off)
                    pltpu.sync_copy(rows_a.at[pl.ds(0, sz)],
                                    acc.at[pl.ds(NSUB * rpw0 + off, sz)])
        pltpu.sync_copy(src_hbm.at[c, s], sidx)
        plsc.subcore_barrier()

        rows = (rows_a, rows_b)
        sems = (sem_a, sem_b)

        def gather(j, parity):
            pltpu.async_copy(g_hbm.at[sidx.at[j]], rows[parity], sems[parity])

        def wait_rows(parity):
            pltpu.make_async_copy(
                g_hbm.at[pl.ds(0, CHUNK)], rows[parity], sems[parity]).wait()

        def fetch_blk(bi, blk, sem):
            off = pl.multiple_of(bi * BLK, BLK)
            pltpu.async_copy(dst_hbm.at[s].at[pl.ds(off, BLK)], blk, sem)

        def wait_blk(blk, sem):
            pltpu.make_async_copy(
                dst_hbm.at[0].at[pl.ds(0, BLK)], blk, sem).wait()

        nb = nch // BLK
        fetch_blk(0, blk_a, sem_da)
        fetch_blk(1, blk_b, sem_db)
        gather(0, 0)

        def body(p, carry):
            j0 = 2 * BLK * p
            wait_blk(blk_a, sem_da)
            for k in range(BLK):
                j = j0 + k
                gather(jnp.minimum(j + 1, nch - 1), (k + 1) % 2)
                wait_rows(k % 2)
                pltpu.sync_copy(rows[k % 2], acc.at[blk_a.at[k]], add=True)
            fetch_blk(jnp.minimum(2 * p + 2, nb - 1), blk_a, sem_da)
            wait_blk(blk_b, sem_db)
            for k in range(BLK):
                j = j0 + BLK + k
                gather(jnp.minimum(j + 1, nch - 1), (k + 1) % 2)
                wait_rows(k % 2)
                pltpu.sync_copy(rows[k % 2], acc.at[blk_b.at[k]], add=True)
            fetch_blk(jnp.minimum(2 * p + 3, nb - 1), blk_b, sem_db)
            return carry

        lax.fori_loop(0, nbp, body, 0)
        wait_rows(0)
        wait_blk(blk_a, sem_da)
        wait_blk(blk_b, sem_db)
        plsc.subcore_barrier()
        base = c * n + s * rpw0
        for off in range(0, rpw0, CHUNK):
            sz = min(CHUNK, rpw0 - off)
            pltpu.sync_copy(acc.at[pl.ds(s * rpw0 + off, sz)],
                            rows_a.at[pl.ds(0, sz)])
            pltpu.sync_copy(rows_a.at[pl.ds(0, sz)],
                            out_hbm.at[pl.ds(base + off, sz)])
        if rem:
            @pl.when(s == NSUB - 1)
            def _():
                pltpu.sync_copy(acc.at[pl.ds(NSUB * rpw0, rem)],
                                rows_a.at[pl.ds(0, rem)])
                pltpu.sync_copy(rows_a.at[pl.ds(0, rem)],
                                out_hbm.at[pl.ds(c * n + NSUB * rpw0, rem)])

    return agg_kernel


def _dinv_of(deg_ref):
    return lax.rsqrt(deg_ref[0, :, 0:1] + deg_ref[1, :, 0:1] + 1.0)


def _k0_body(x_ref, wpre_ref, bpre_ref, h_ref):
    h_ref[...] = jnp.maximum(
        jnp.dot(x_ref[...], wpre_ref[...], preferred_element_type=jnp.float32)
        + bpre_ref[...], 0.0)


def _k1_body(h_ref, deg_ref, w1_ref, g_ref):
    dinv = _dinv_of(deg_ref)
    g = jnp.dot(h_ref[...], w1_ref[...],
                preferred_element_type=jnp.float32) * dinv
    g_ref[0] = g[:, :128]
    g_ref[1] = g[:, 128:]


def _kmid_body(h_ref, acc_ref, deg_ref, b_ref, w_ref, hn_ref, g_ref):
    dinv = _dinv_of(deg_ref)
    agg = jnp.concatenate([acc_ref[0], acc_ref[1]], axis=1)
    hn = h_ref[...] + jnp.maximum(agg * dinv + b_ref[...], 0.0)
    hn_ref[...] = hn
    g = jnp.dot(hn, w_ref[...], preferred_element_type=jnp.float32) * dinv
    g_ref[0] = g[:, :128]
    g_ref[1] = g[:, 128:]


def _kfin_body(h_ref, acc_ref, deg_ref, b_ref, wh_ref, bh_ref, y_ref):
    dinv = _dinv_of(deg_ref)
    agg = jnp.concatenate([acc_ref[0], acc_ref[1]], axis=1)
    hn = h_ref[...] + jnp.maximum(agg * dinv + b_ref[...], 0.0)
    y_ref[...] = (
        jnp.dot(hn, wh_ref[...], preferred_element_type=jnp.float32)
        + bh_ref[...])


def _row_spec(d):
    return pl.BlockSpec((BR, d), lambda i: (i, 0))


def _full_spec(shape):
    nd = len(shape)
    return pl.BlockSpec(shape, lambda i, _n=nd: (0,) * _n)


def _split_spec():
    return pl.BlockSpec((2, BR, 128), lambda i: (0, i, 0))


def _deg_spec():
    return pl.BlockSpec((2, BR, 128), lambda i: (0, i, 0))


def kernel(x, edge_index, W_pre, b_pre, W1, b1, W2, b2, W3, b3, W_head, b_head):
    n, d = x.shape
    e = edge_index.shape[1]
    dout = W_head.shape[1]
    grid = (n // BR,)

    src = edge_index[0]
    dst = edge_index[1]

    loop = jnp.arange(n, dtype=src.dtype)
    srcl = jnp.concatenate([src, loop])
    dstl = jnp.concatenate([dst, loop])
    e2 = e + n
    nch = _cdiv(_cdiv(e2, NSUB * CHUNK), 2 * BLK) * 2 * BLK
    cap = NSUB * CHUNK * nch
    dsta = jnp.pad(dstl, (0, cap - e2), constant_values=n)
    dsta = dsta.reshape(NSUB, nch, CHUNK)
    srcp = jnp.pad(srcl, (0, cap - e2)).reshape(NSUB, nch, CHUNK)
    srca = jnp.stack([srcp, srcp + n], axis=0)

    zeros128 = jnp.zeros((CHUNK, 128), jnp.float32)
    ones128 = jnp.ones((CHUNK, 128), jnp.float32)

    agg = _make_agg(n, nch)

    nch3 = _cdiv(_cdiv(e, NCORE * NSUB * CHUNK), 2 * BLK) * 2 * BLK
    cap3 = NCORE * NSUB * CHUNK * nch3
    dst3 = jnp.pad(dst, (0, cap3 - e), constant_values=n)
    dst3 = dst3.reshape(NCORE, NSUB, nch3, CHUNK)
    deg = _make_deg(n, nch3)(dst3, ones128, zeros128).reshape(NCORE, n, 128)

    b_pre2 = b_pre.reshape(1, d)
    b1_2 = b1.reshape(1, d)
    b2_2 = b2.reshape(1, d)
    b3_2 = b3.reshape(1, d)
    bh_2 = b_head.reshape(1, dout)

    k0 = pl.pallas_call(
        _k0_body,
        grid=grid,
        in_specs=[_row_spec(d), _full_spec((d, d)), _full_spec((1, d))],
        out_specs=_row_spec(d),
        out_shape=jax.ShapeDtypeStruct((n, d), jnp.float32),
    )
    k1 = pl.pallas_call(
        _k1_body,
        grid=grid,
        in_specs=[_row_spec(d), _deg_spec(), _full_spec((d, d))],
        out_specs=_split_spec(),
        out_shape=jax.ShapeDtypeStruct((2, n, 128), jnp.float32),
    )
    kmid = pl.pallas_call(
        _kmid_body,
        grid=grid,
        in_specs=[_row_spec(d), _split_spec(), _deg_spec(),
                  _full_spec((1, d)), _full_spec((d, d))],
        out_specs=[_row_spec(d), _split_spec()],
        out_shape=[jax.ShapeDtypeStruct((n, d), jnp.float32),
                   jax.ShapeDtypeStruct((2, n, 128), jnp.float32)],
    )
    kfin = pl.pallas_call(
        _kfin_body,
        grid=grid,
        in_specs=[_row_spec(d), _split_spec(), _deg_spec(),
                  _full_spec((1, d)), _full_spec((d, dout)),
                  _full_spec((1, dout))],
        out_specs=_row_spec(dout),
        out_shape=jax.ShapeDtypeStruct((n, dout), jnp.float32),
    )

    h0 = k0(x, W_pre, b_pre2)
    g1 = k1(h0, deg, W1)
    acc1 = agg(g1.reshape(2 * n, 128), srca, dsta, zeros128).reshape(2, n, 128)
    h1, g2 = kmid(h0, acc1, deg, b1_2, W2)
    acc2 = agg(g2.reshape(2 * n, 128), srca, dsta, zeros128).reshape(2, n, 128)
    h2, g3 = kmid(h1, acc2, deg, b2_2, W3)
    acc3 = agg(g3.reshape(2 * n, 128), srca, dsta, zeros128).reshape(2, n, 128)
    return kfin(h2, acc3, deg, b3_2, W_head, bh_2)


# speedup vs baseline: 9.2016x; 1.0014x over previous
import functools

import jax
import jax.numpy as jnp
from jax import lax
from jax.experimental import pallas as pl
from jax.experimental.pallas import tpu as pltpu
from jax.experimental.pallas import tpu_sc as plsc

NCORE = 2
NSUB = 16
CHUNK = 112
BLK = 8
ACHUNK = 112
ABLK = 8
ARING = 2
BR = 1000


def _cdiv(a, b):
    return (a + b - 1) // b


def _splits(n):
    rpw0 = (n // NSUB) // 8 * 8
    rem = n - NSUB * rpw0
    assert rem % 8 == 0
    return rpw0, rem


@functools.lru_cache(maxsize=None)
def _make_deg(n, nch):
    mesh = plsc.VectorSubcoreMesh(
        core_axis_name="c", subcore_axis_name="s",
        num_cores=NCORE, num_subcores=NSUB)
    rpw0, rem = _splits(n)
    assert nch % (2 * BLK) == 0
    nbp = nch // (2 * BLK)
    npad = (n + 16 + 7) // 8 * 8
    zrem = npad - NSUB * rpw0

    @functools.partial(
        pl.kernel,
        out_type=jax.ShapeDtypeStruct((NCORE * n, 128), jnp.float32),
        mesh=mesh,
        scratch_types=[
            pltpu.VMEM((BLK, CHUNK), jnp.int32),
            pltpu.VMEM((BLK, CHUNK), jnp.int32),
            pltpu.VMEM((CHUNK, 128), jnp.float32),
            pltpu.VMEM_SHARED((npad, 128), jnp.float32),
            pltpu.SemaphoreType.DMA,
            pltpu.SemaphoreType.DMA,
        ],
    )
    def deg_kernel(dst_hbm, ones_hbm, zeros_hbm, out_hbm,
                   blk_a, blk_b, ones_v, acc, sem_da, sem_db):
        c = lax.axis_index("c")
        s = lax.axis_index("s")
        pltpu.sync_copy(zeros_hbm, ones_v)
        for off in range(0, rpw0, CHUNK):
            sz = min(CHUNK, rpw0 - off)
            pltpu.sync_copy(ones_v.at[pl.ds(0, sz)],
                            acc.at[pl.ds(s * rpw0 + off, sz)])
        if zrem:
            @pl.when(s == NSUB - 1)
            def _():
                for off in range(0, zrem, CHUNK):
                    sz = min(CHUNK, zrem - off)
                    pltpu.sync_copy(ones_v.at[pl.ds(0, sz)],
                                    acc.at[pl.ds(NSUB * rpw0 + off, sz)])
        pltpu.sync_copy(ones_hbm, ones_v)
        plsc.subcore_barrier()

        def fetch_blk(bi, blk, sem):
            off = pl.multiple_of(bi * BLK, BLK)
            pltpu.async_copy(dst_hbm.at[c, s].at[pl.ds(off, BLK)], blk, sem)

        def wait_blk(blk, sem):
            pltpu.make_async_copy(
                dst_hbm.at[0, 0].at[pl.ds(0, BLK)], blk, sem).wait()

        nb = nch // BLK
        fetch_blk(0, blk_a, sem_da)
        fetch_blk(1, blk_b, sem_db)

        def body(p, carry):
            wait_blk(blk_a, sem_da)
            for k in range(BLK):
                pltpu.sync_copy(ones_v, acc.at[blk_a.at[k]], add=True)
            fetch_blk(jnp.minimum(2 * p + 2, nb - 1), blk_a, sem_da)
            wait_blk(blk_b, sem_db)
            for k in range(BLK):
                pltpu.sync_copy(ones_v, acc.at[blk_b.at[k]], add=True)
            fetch_blk(jnp.minimum(2 * p + 3, nb - 1), blk_b, sem_db)
            return carry

        lax.fori_loop(0, nbp, body, 0)
        wait_blk(blk_a, sem_da)
        wait_blk(blk_b, sem_db)
        plsc.subcore_barrier()
        base = c * n + s * rpw0
        for off in range(0, rpw0, CHUNK):
            sz = min(CHUNK, rpw0 - off)
            pltpu.sync_copy(acc.at[pl.ds(s * rpw0 + off, sz)],
                            ones_v.at[pl.ds(0, sz)])
            pltpu.sync_copy(ones_v.at[pl.ds(0, sz)],
                            out_hbm.at[pl.ds(base + off, sz)])
        if rem:
            @pl.when(s == NSUB - 1)
            def _():
                pltpu.sync_copy(acc.at[pl.ds(NSUB * rpw0, rem)],
                                ones_v.at[pl.ds(0, rem)])
                pltpu.sync_copy(ones_v.at[pl.ds(0, rem)],
                                out_hbm.at[pl.ds(c * n + NSUB * rpw0, rem)])

    return deg_kernel


@functools.lru_cache(maxsize=None)
def _make_agg(n, nch, chunk, blk, ring):
    mesh = plsc.VectorSubcoreMesh(
        core_axis_name="c", subcore_axis_name="s",
        num_cores=NCORE, num_subcores=NSUB)
    rpw0, rem = _splits(n)
    nbody = 2 * blk
    P = ring - 1
    assert nbody % ring == 0 and nch % nbody == 0 and P < blk
    nbp = nch // nbody
    nb = nch // blk
    npad = (n + 16 + 7) // 8 * 8
    zrem = npad - NSUB * rpw0

    @functools.partial(
        pl.kernel,
        out_type=jax.ShapeDtypeStruct((NCORE * n, 128), jnp.float32),
        mesh=mesh,
        scratch_types=(
            [pltpu.VMEM((blk, chunk), jnp.int32)] * 4
            + [pltpu.VMEM((chunk, 128), jnp.float32)] * ring
            + [pltpu.VMEM_SHARED((npad, 128), jnp.float32)]
            + [pltpu.SemaphoreType.DMA] * (4 + ring)
        ),
    )
    def agg_kernel(g_hbm, src_hbm, dst_hbm, zeros_hbm, out_hbm, *scr):
        sblk = scr[0:2]
        dblk = scr[2:4]
        rows = scr[4:4 + ring]
        acc = scr[4 + ring]
        bsem = scr[5 + ring:9 + ring]
        rsem = scr[9 + ring:]
        c = lax.axis_index("c")
        s = lax.axis_index("s")
        pltpu.sync_copy(zeros_hbm, rows[0])
        for off in range(0, rpw0, chunk):
            sz = min(chunk, rpw0 - off)
            pltpu.sync_copy(rows[0].at[pl.ds(0, sz)],
                            acc.at[pl.ds(s * rpw0 + off, sz)])
        if zrem:
            @pl.when(s == NSUB - 1)
            def _():
                for off in range(0, zrem, chunk):
                    sz = min(chunk, zrem - off)
                    pltpu.sync_copy(rows[0].at[pl.ds(0, sz)],
                                    acc.at[pl.ds(NSUB * rpw0 + off, sz)])
        plsc.subcore_barrier()

        def fetch_sblk(bi, i):
            off = pl.multiple_of(bi * blk, blk)
            pltpu.async_copy(src_hbm.at[c, s].at[pl.ds(off, blk)],
                             sblk[i], bsem[i])

        def wait_sblk(i):
            pltpu.make_async_copy(src_hbm.at[0, 0].at[pl.ds(0, blk)],
                                  sblk[i], bsem[i]).wait()

        def fetch_dblk(bi, i):
            off = pl.multiple_of(bi * blk, blk)
            pltpu.async_copy(dst_hbm.at[s].at[pl.ds(off, blk)],
                             dblk[i], bsem[2 + i])

        def wait_dblk(i):
            pltpu.make_async_copy(dst_hbm.at[0].at[pl.ds(0, blk)],
                                  dblk[i], bsem[2 + i]).wait()

        def gather(sidx, r):
            pltpu.async_copy(g_hbm.at[sidx], rows[r], rsem[r])

        def wait_rows(r):
            pltpu.make_async_copy(g_hbm.at[pl.ds(0, chunk)],
                                  rows[r], rsem[r]).wait()

        fetch_sblk(0, 0)
        wait_sblk(0)
        fetch_dblk(0, 0)
        fetch_sblk(1, 1)
        fetch_dblk(1, 1)
        for k in range(P):
            gather(sblk[0].at[k], k)

        def body(p, carry):
            j0 = nbody * p
            wait_dblk(0)
            for k in range(nbody):
                if k == blk:
                    wait_dblk(1)
                if k == blk - P:
                    wait_sblk(1)
                if k == nbody - P:
                    wait_sblk(0)
                kp = k + P
                if kp < nbody:
                    gather(sblk[kp // blk].at[kp % blk], kp % ring)
                else:
                    jn = jnp.minimum(j0 + kp, nch - 1) - j0 - nbody
                    jn = jnp.maximum(jn, 0)
                    gather(sblk[0].at[jn], kp % ring)
                wait_rows(k % ring)
                pltpu.sync_copy(rows[k % ring],
                                acc.at[dblk[k // blk].at[k % blk]], add=True)
                if k == blk - 1:
                    fetch_sblk(jnp.minimum(2 * p + 2, nb - 1), 0)
                    fetch_dblk(jnp.minimum(2 * p + 2, nb - 1), 0)
                if k == nbody - 1:
                    fetch_sblk(jnp.minimum(2 * p + 3, nb - 1), 1)
            fetch_dblk(jnp.minimum(2 * p + 3, nb - 1), 1)
            return carry

        lax.fori_loop(0, nbp, body, 0)
        for t in range(P):
            wait_rows(t % ring)
        wait_sblk(1)
        wait_dblk(0)
        wait_dblk(1)
        plsc.subcore_barrier()
        base = c * n + s * rpw0
        for off in range(0, rpw0, chunk):
            sz = min(chunk, rpw0 - off)
            pltpu.sync_copy(acc.at[pl.ds(s * rpw0 + off, sz)],
                            rows[0].at[pl.ds(0, sz)])
            pltpu.sync_copy(rows[0].at[pl.ds(0, sz)],
                            out_hbm.at[pl.ds(base + off, sz)])
        if rem:
            @pl.when(s == NSUB - 1)
            def _():
                pltpu.sync_copy(acc.at[pl.ds(NSUB * rpw0, rem)],
                                rows[0].at[pl.ds(0, rem)])
                pltpu.sync_copy(rows[0].at[pl.ds(0, rem)],
                                out_hbm.at[pl.ds(c * n + NSUB * rpw0, rem)])

    return agg_kernel


def _dinv_of(deg_ref):
    return lax.rsqrt(deg_ref[0, :, 0:1] + deg_ref[1, :, 0:1] + 1.0)


def _k0_body(x_ref, wpre_ref, bpre_ref, h_ref):
    h_ref[...] = jnp.maximum(
        jnp.dot(x_ref[...], wpre_ref[...], preferred_element_type=jnp.float32)
        + bpre_ref[...], 0.0)


def _k1_body(h_ref, deg_ref, w1_ref, g_ref):
    dinv = _dinv_of(deg_ref)
    g = jnp.dot(h_ref[...], w1_ref[...],
                preferred_element_type=jnp.float32) * dinv
    g_ref[0] = g[:, :128]
    g_ref[1] = g[:, 128:]


def _kmid_body(h_ref, acc_ref, deg_ref, b_ref, w_ref, hn_ref, g_ref):
    dinv = _dinv_of(deg_ref)
    agg = jnp.concatenate([acc_ref[0], acc_ref[1]], axis=1)
    hn = h_ref[...] + jnp.maximum(agg * dinv + b_ref[...], 0.0)
    hn_ref[...] = hn
    g = jnp.dot(hn, w_ref[...], preferred_element_type=jnp.float32) * dinv
    g_ref[0] = g[:, :128]
    g_ref[1] = g[:, 128:]


def _kfin_body(h_ref, acc_ref, deg_ref, b_ref, wh_ref, bh_ref, y_ref):
    dinv = _dinv_of(deg_ref)
    agg = jnp.concatenate([acc_ref[0], acc_ref[1]], axis=1)
    hn = h_ref[...] + jnp.maximum(agg * dinv + b_ref[...], 0.0)
    y_ref[...] = (
        jnp.dot(hn, wh_ref[...], preferred_element_type=jnp.float32)
        + bh_ref[...])


def _row_spec(d):
    return pl.BlockSpec((BR, d), lambda i: (i, 0))


def _full_spec(shape):
    nd = len(shape)
    return pl.BlockSpec(shape, lambda i, _n=nd: (0,) * _n)


def _split_spec():
    return pl.BlockSpec((2, BR, 128), lambda i: (0, i, 0))


def _deg_spec():
    return pl.BlockSpec((2, BR, 128), lambda i: (0, i, 0))


def kernel(x, edge_index, W_pre, b_pre, W1, b1, W2, b2, W3, b3, W_head, b_head):
    n, d = x.shape
    e = edge_index.shape[1]
    dout = W_head.shape[1]
    grid = (n // BR,)

    src = edge_index[0]
    dst = edge_index[1]

    loop = jnp.arange(n, dtype=src.dtype)
    srcl = jnp.concatenate([src, loop])
    dstl = jnp.concatenate([dst, loop])
    e2 = e + n
    nch = _cdiv(_cdiv(e2, NSUB * ACHUNK), 2 * ABLK) * 2 * ABLK
    cap = NSUB * ACHUNK * nch
    dsta = jnp.pad(dstl, (0, cap - e2), constant_values=n)
    dsta = dsta.reshape(NSUB, nch, ACHUNK)
    srcp = jnp.pad(srcl, (0, cap - e2)).reshape(NSUB, nch, ACHUNK)
    srca = jnp.stack([srcp, srcp + n], axis=0)

    zeros128 = jnp.zeros((CHUNK, 128), jnp.float32)
    zeros_a = jnp.zeros((ACHUNK, 128), jnp.float32)
    ones128 = jnp.ones((CHUNK, 128), jnp.float32)

    agg = _make_agg(n, nch, ACHUNK, ABLK, ARING)

    nch3 = _cdiv(_cdiv(e, NCORE * NSUB * CHUNK), 2 * BLK) * 2 * BLK
    cap3 = NCORE * NSUB * CHUNK * nch3
    dst3 = jnp.pad(dst, (0, cap3 - e), constant_values=n)
    dst3 = dst3.reshape(NCORE, NSUB, nch3, CHUNK)
    deg = _make_deg(n, nch3)(dst3, ones128, zeros128).reshape(NCORE, n, 128)

    b_pre2 = b_pre.reshape(1, d)
    b1_2 = b1.reshape(1, d)
    b2_2 = b2.reshape(1, d)
    b3_2 = b3.reshape(1, d)
    bh_2 = b_head.reshape(1, dout)

    k0 = pl.pallas_call(
        _k0_body,
        grid=grid,
        in_specs=[_row_spec(d), _full_spec((d, d)), _full_spec((1, d))],
        out_specs=_row_spec(d),
        out_shape=jax.ShapeDtypeStruct((n, d), jnp.float32),
    )
    k1 = pl.pallas_call(
        _k1_body,
        grid=grid,
        in_specs=[_row_spec(d), _deg_spec(), _full_spec((d, d))],
        out_specs=_split_spec(),
        out_shape=jax.ShapeDtypeStruct((2, n, 128), jnp.float32),
    )
    kmid = pl.pallas_call(
        _kmid_body,
        grid=grid,
        in_specs=[_row_spec(d), _split_spec(), _deg_spec(),
                  _full_spec((1, d)), _full_spec((d, d))],
        out_specs=[_row_spec(d), _split_spec()],
        out_shape=[jax.ShapeDtypeStruct((n, d), jnp.float32),
                   jax.ShapeDtypeStruct((2, n, 128), jnp.float32)],
    )
    kfin = pl.pallas_call(
        _kfin_body,
        grid=grid,
        in_specs=[_row_spec(d), _split_spec(), _deg_spec(),
                  _full_spec((1, d)), _full_spec((d, dout)),
                  _full_spec((1, dout))],
        out_specs=_row_spec(dout),
        out_shape=jax.ShapeDtypeStruct((n, dout), jnp.float32),
    )

    h0 = k0(x, W_pre, b_pre2)
    g1 = k1(h0, deg, W1)
    acc1 = agg(g1.reshape(2 * n, 128), srca, dsta, zeros_a).reshape(2, n, 128)
    h1, g2 = kmid(h0, acc1, deg, b1_2, W2)
    acc2 = agg(g2.reshape(2 * n, 128), srca, dsta, zeros_a).reshape(2, n, 128)
    h2, g3 = kmid(h1, acc2, deg, b2_2, W3)
    acc3 = agg(g3.reshape(2 * n, 128), srca, dsta, zeros_a).reshape(2, n, 128)
    return kfin(h2, acc3, deg, b3_2, W_head, bh_2)
